# baseline (speedup 1.0000x reference)
import functools

import jax
import jax.numpy as jnp
from jax import lax
from jax.experimental import pallas as pl
from jax.experimental.pallas import tpu as pltpu

_NSLOT = 4
_CH_ROWS = 512


def _mil_body(inv_ref, x_hbm, p_hbm, n_hbm, out_ref, xb, pb, nb, s0, s1, s2, s3):
    n_chunks = x_hbm.shape[0] // _CH_ROWS
    sems = (s0, s1, s2, s3)
    bufs = ((xb, x_hbm), (pb, p_hbm), (nb, n_hbm))

    def copies(ci, k):
        return [
            pltpu.make_async_copy(hbm.at[pl.ds(ci * _CH_ROWS, _CH_ROWS)], buf.at[k], sems[k])
            for (buf, hbm) in bufs
        ]

    for k in range(_NSLOT):
        for c in copies(k, k):
            c.start()

    def body(ci, total):
        slot = lax.rem(ci, _NSLOT)
        for k in range(_NSLOT):
            @pl.when(slot == k)
            def _():
                for c in copies(ci, k):
                    c.wait()

        xv = xb[slot]
        m = (pb[slot] > 0.5) & (nb[slot] > 0.5)
        mf = m.astype(jnp.float32)
        probs = jax.nn.sigmoid(xv)
        ps = jnp.sum(probs * mf)
        pc = jnp.sum(mf)

        for k in range(_NSLOT):
            @pl.when(jnp.logical_and(slot == k, ci + _NSLOT < n_chunks))
            def _():
                for c in copies(ci + _NSLOT, k):
                    c.start()

        p = ps / pc
        inv = inv_ref[ci]
        return total + (-inv * jnp.log(p) - (1.0 - inv) * jnp.log(1.0 - p))

    total = lax.fori_loop(0, n_chunks, body, jnp.float32(0.0))
    out_ref[...] = total.reshape(1, 1)


def kernel(cancer_logits, prostate_mask, needle_mask, involvement, grade_group):
    B, _, H, W = cancer_logits.shape
    x = cancer_logits.reshape(B * H, W)
    pm = prostate_mask.reshape(B * H, W)
    nm = needle_mask.reshape(B * H, W)

    out = pl.pallas_call(
        _mil_body,
        in_specs=[
            pl.BlockSpec(memory_space=pltpu.SMEM),
            pl.BlockSpec(memory_space=pl.ANY),
            pl.BlockSpec(memory_space=pl.ANY),
            pl.BlockSpec(memory_space=pl.ANY),
        ],
        out_specs=pl.BlockSpec(memory_space=pltpu.VMEM),
        out_shape=jax.ShapeDtypeStruct((1, 1), jnp.float32),
        scratch_shapes=[
            pltpu.VMEM((_NSLOT, _CH_ROWS, W), jnp.float32),
            pltpu.VMEM((_NSLOT, _CH_ROWS, W), jnp.float32),
            pltpu.VMEM((_NSLOT, _CH_ROWS, W), jnp.float32),
            pltpu.SemaphoreType.DMA,
            pltpu.SemaphoreType.DMA,
            pltpu.SemaphoreType.DMA,
            pltpu.SemaphoreType.DMA,
        ],
    )(involvement, x, pm, nm)
    return out[0, 0]


# speedup vs baseline: 1.0384x; 1.0384x over previous
import functools

import jax
import jax.numpy as jnp
from jax import lax
from jax.experimental import pallas as pl
from jax.experimental.pallas import tpu as pltpu

_NSLOT = 4
_CH_ROWS = 512


def _mil_body(inv_ref, x_hbm, p_hbm, n_hbm, out_ref, xb, pb, nb, s0, s1, s2, s3):
    n_chunks = x_hbm.shape[0] // _CH_ROWS
    sems = (s0, s1, s2, s3)
    bufs = ((xb, x_hbm), (pb, p_hbm), (nb, n_hbm))

    def copies(ci, k):
        return [
            pltpu.make_async_copy(hbm.at[pl.ds(ci * _CH_ROWS, _CH_ROWS)], buf.at[k], sems[k])
            for (buf, hbm) in bufs
        ]

    for k in range(_NSLOT - 1):
        for c in copies(k, k):
            c.start()

    def body(ci, total):
        slot = lax.rem(ci, _NSLOT)
        for k in range(_NSLOT):
            @pl.when(slot == k)
            def _():
                for c in copies(ci, k):
                    c.wait()

        nxt = lax.rem(ci + _NSLOT - 1, _NSLOT)
        for k in range(_NSLOT):
            @pl.when(jnp.logical_and(nxt == k, ci + _NSLOT - 1 < n_chunks))
            def _():
                for c in copies(ci + _NSLOT - 1, k):
                    c.start()

        xv = xb[slot]
        m = (pb[slot] > 0.5) & (nb[slot] > 0.5)
        mf = m.astype(jnp.float32)
        probs = jax.nn.sigmoid(xv)
        ps = jnp.sum(probs * mf)
        pc = jnp.sum(mf)

        p = ps / pc
        inv = inv_ref[ci]
        return total + (-inv * jnp.log(p) - (1.0 - inv) * jnp.log(1.0 - p))

    total = lax.fori_loop(0, n_chunks, body, jnp.float32(0.0))
    out_ref[...] = total.reshape(1, 1)


def kernel(cancer_logits, prostate_mask, needle_mask, involvement, grade_group):
    B, _, H, W = cancer_logits.shape
    x = cancer_logits.reshape(B * H, W)
    pm = prostate_mask.reshape(B * H, W)
    nm = needle_mask.reshape(B * H, W)

    out = pl.pallas_call(
        _mil_body,
        in_specs=[
            pl.BlockSpec(memory_space=pltpu.SMEM),
            pl.BlockSpec(memory_space=pl.ANY),
            pl.BlockSpec(memory_space=pl.ANY),
            pl.BlockSpec(memory_space=pl.ANY),
        ],
        out_specs=pl.BlockSpec(memory_space=pltpu.VMEM),
        out_shape=jax.ShapeDtypeStruct((1, 1), jnp.float32),
        scratch_shapes=[
            pltpu.VMEM((_NSLOT, _CH_ROWS, W), jnp.float32),
            pltpu.VMEM((_NSLOT, _CH_ROWS, W), jnp.float32),
            pltpu.VMEM((_NSLOT, _CH_ROWS, W), jnp.float32),
            pltpu.SemaphoreType.DMA,
            pltpu.SemaphoreType.DMA,
            pltpu.SemaphoreType.DMA,
            pltpu.SemaphoreType.DMA,
        ],
    )(involvement, x, pm, nm)
    return out[0, 0]
